# baseline (speedup 1.0000x reference)
import functools

import jax
import jax.numpy as jnp
from jax import lax
from jax.experimental import pallas as pl
from jax.experimental.pallas import tpu as pltpu
from jax.experimental.pallas import tpu_sc as plsc

_BB = 128


@functools.cache
def _make_gather(V, D, BATCH, HIST):
    info = plsc.get_sparse_core_info()
    NC, NS, L = info.num_cores, info.num_subcores, info.num_lanes
    NW = NC * NS
    assert BATCH % (NW * _BB) == 0 and D % 8 == 0
    b_w = BATCH // NW
    njb = b_w // _BB
    n_units = HIST * njb
    assert n_units % 2 == 0
    DB = D // 8
    mesh = plsc.VectorSubcoreMesh(core_axis_name="c", subcore_axis_name="s")

    @functools.partial(
        pl.kernel,
        mesh=mesh,
        out_type=jax.ShapeDtypeStruct((HIST, DB, BATCH // _BB, 8, _BB), jnp.float32),
        scratch_types=[
            pltpu.VMEM((b_w, HIST), jnp.int32),
            pltpu.VMEM((HIST, b_w), jnp.int32),
            [pltpu.VMEM((_BB, D), jnp.float32) for _ in range(2)],
            [pltpu.VMEM((DB, 8, _BB), jnp.float32) for _ in range(2)],
            [pltpu.SemaphoreType.DMA for _ in range(2)],
            [pltpu.SemaphoreType.DMA for _ in range(2)],
        ],
        compiler_params=pltpu.CompilerParams(
            use_tc_tiling_on_sc=False, needs_layout_passes=False
        ),
    )
    def gather_kernel(feat_hbm, table_hbm, out_hbm, idx_v, idx_t, grows, tbufs,
                      gsems, ssems):
        wid = lax.axis_index("s") * NC + lax.axis_index("c")
        base = wid * b_w
        lanes = lax.iota(jnp.int32, L)
        pltpu.sync_copy(feat_hbm.at[pl.ds(base, b_w)], idx_v)

        def tr_idx(h, carry):
            for k in range(b_w // L):
                v = plsc.load_gather(
                    idx_v, [lanes + k * L, jnp.full((L,), h, jnp.int32)])
                idx_t[h, pl.ds(k * L, L)] = v
            return carry

        lax.fori_loop(0, HIST, tr_idx, 0)

        def start_gather(u, p):
            h, jb = u // njb, u % njb
            return pltpu.async_copy(
                table_hbm.at[idx_t.at[h, pl.ds(jb * _BB, _BB)]], grows[p],
                gsems[p])

        def wait_gather(p):
            pltpu.make_async_copy(
                table_hbm.at[idx_t.at[0, pl.ds(0, _BB)]], grows[p], gsems[p]
            ).wait()

        def transpose_unit(p):
            def tr(db, carry):
                for ds in range(8):
                    col = jnp.full((L,), db * 8 + ds, jnp.int32)
                    for k in range(_BB // L):
                        v = plsc.load_gather(grows[p], [lanes + k * L, col])
                        tbufs[p][db, ds, pl.ds(k * L, L)] = v
                return carry

            lax.fori_loop(0, DB, tr, 0)

        def start_store(u, p):
            h, jb = u // njb, u % njb
            return pltpu.async_copy(
                tbufs[p], out_hbm.at[h, :, wid * njb + jb], ssems[p])

        def wait_store(p):
            pltpu.make_async_copy(
                tbufs[p], out_hbm.at[0, :, 0], ssems[p]).wait()

        start_gather(0, 0)

        def pair_body(q, carry):
            u0 = 2 * q
            wait_gather(0)
            start_gather(u0 + 1, 1)
            transpose_unit(0)
            lax.cond(q > 0, lambda: wait_store(0), lambda: None)
            start_store(u0, 0)
            wait_gather(1)
            lax.cond(u0 + 2 < n_units,
                     lambda: (start_gather(u0 + 2, 0), None)[1], lambda: None)
            transpose_unit(1)
            lax.cond(q > 0, lambda: wait_store(1), lambda: None)
            start_store(u0 + 1, 1)
            return carry

        lax.fori_loop(0, n_units // 2, pair_body, 0)
        wait_store(0)
        wait_store(1)

    return gather_kernel


def kernel(feature, table):
    batch, hist = feature.shape
    dim = table.shape[1]
    out_p = _make_gather(table.shape[0], dim, batch, hist)(feature, table)
    return jnp.transpose(out_p, (2, 4, 0, 1, 3)).reshape(batch, hist, dim)


# speedup vs baseline: 1.8388x; 1.8388x over previous
import functools

import jax
import jax.numpy as jnp
from jax import lax
from jax.experimental import pallas as pl
from jax.experimental.pallas import tpu as pltpu
from jax.experimental.pallas import tpu_sc as plsc

_BB = 128


@functools.cache
def _make_gather(V, D, BATCH, HIST):
    info = plsc.get_sparse_core_info()
    NC, NS, L = info.num_cores, info.num_subcores, info.num_lanes
    NW = NC * NS
    assert BATCH % (NW * _BB) == 0 and D % L == 0
    b_w = BATCH // NW
    njb = b_w // _BB
    n_units = HIST * njb
    assert n_units % 2 == 0
    DB = D // 8
    HP = (HIST + L - 1) // L * L
    mesh = plsc.VectorSubcoreMesh(core_axis_name="c", subcore_axis_name="s")

    @functools.partial(
        pl.kernel,
        mesh=mesh,
        out_type=jax.ShapeDtypeStruct((HIST, DB, BATCH // _BB, 8, _BB), jnp.float32),
        scratch_types=[
            pltpu.VMEM((b_w, HIST), jnp.int32),
            pltpu.VMEM((HP, b_w + 1), jnp.int32),
            [pltpu.VMEM((_BB, D), jnp.float32) for _ in range(2)],
            [pltpu.VMEM((DB, 8, _BB + 1), jnp.float32) for _ in range(2)],
            [pltpu.SemaphoreType.DMA for _ in range(2)],
            [pltpu.SemaphoreType.DMA for _ in range(2)],
        ],
        compiler_params=pltpu.CompilerParams(
            use_tc_tiling_on_sc=False, needs_layout_passes=False
        ),
    )
    def gather_kernel(feat_hbm, table_hbm, out_hbm, idx_v, idx_t, grows, tbufs,
                      gsems, ssems):
        wid = lax.axis_index("s") * NC + lax.axis_index("c")
        base = wid * b_w
        lanes = lax.iota(jnp.int32, L)
        pltpu.sync_copy(feat_hbm.at[pl.ds(base, b_w)], idx_v)

        offs = [k * L for k in range(HIST // L)]
        if HIST % L:
            offs.append(HIST - L)

        def tr_idx(b, carry):
            bvec = jnp.full((L,), b, jnp.int32)
            for o in offs:
                v = idx_v[b, pl.ds(o, L)]
                plsc.store_scatter(idx_t, [o + lanes, bvec], v)
            return carry

        lax.fori_loop(0, b_w, tr_idx, 0)

        dbvs = [(k * L + lanes) // 8 for k in range(D // L)]
        dsvs = [(k * L + lanes) % 8 for k in range(D // L)]

        def start_gather(u, p):
            h, jb = u // njb, u % njb
            return pltpu.async_copy(
                table_hbm.at[idx_t.at[h, pl.ds(jb * _BB, _BB)]], grows[p],
                gsems[p])

        def wait_gather(p):
            pltpu.make_async_copy(
                table_hbm.at[idx_t.at[0, pl.ds(0, _BB)]], grows[p], gsems[p]
            ).wait()

        def transpose_unit(p):
            def tr(i, carry):
                for bb in range(4):
                    b = i * 4 + bb
                    bvec = jnp.full((L,), b, jnp.int32)
                    for k in range(D // L):
                        v = grows[p][b, pl.ds(k * L, L)]
                        plsc.store_scatter(tbufs[p], [dbvs[k], dsvs[k], bvec], v)
                return carry

            lax.fori_loop(0, _BB // 4, tr, 0)

        def start_store(u, p):
            h, jb = u // njb, u % njb
            return pltpu.async_copy(
                tbufs[p].at[:, :, pl.ds(0, _BB)],
                out_hbm.at[h, :, wid * njb + jb], ssems[p])

        def wait_store(p):
            pltpu.make_async_copy(
                tbufs[p].at[:, :, pl.ds(0, _BB)], out_hbm.at[0, :, 0], ssems[p]
            ).wait()

        start_gather(0, 0)

        def pair_body(q, carry):
            u0 = 2 * q
            wait_gather(0)
            start_gather(u0 + 1, 1)
            transpose_unit(0)
            lax.cond(q > 0, lambda: wait_store(0), lambda: None)
            start_store(u0, 0)
            wait_gather(1)
            lax.cond(u0 + 2 < n_units,
                     lambda: (start_gather(u0 + 2, 0), None)[1], lambda: None)
            transpose_unit(1)
            lax.cond(q > 0, lambda: wait_store(1), lambda: None)
            start_store(u0 + 1, 1)
            return carry

        lax.fori_loop(0, n_units // 2, pair_body, 0)
        wait_store(0)
        wait_store(1)

    return gather_kernel


def kernel(feature, table):
    batch, hist = feature.shape
    dim = table.shape[1]
    out_p = _make_gather(table.shape[0], dim, batch, hist)(feature, table)
    return jnp.transpose(out_p, (2, 4, 0, 1, 3)).reshape(batch, hist, dim)


# speedup vs baseline: 1.8492x; 1.0056x over previous
import functools

import jax
import jax.numpy as jnp
from jax import lax
from jax.experimental import pallas as pl
from jax.experimental.pallas import tpu as pltpu
from jax.experimental.pallas import tpu_sc as plsc

_BB = 128


@functools.cache
def _make_gather(V, D, BATCH, HIST):
    info = plsc.get_sparse_core_info()
    NC, NS, L = info.num_cores, info.num_subcores, info.num_lanes
    NW = NC * NS
    assert BATCH % (NW * _BB) == 0 and D % L == 0
    b_w = BATCH // NW
    njb = b_w // _BB
    n_units = HIST * njb
    assert n_units % 2 == 0
    DB = D // 8
    HP = (HIST + L - 1) // L * L
    mesh = plsc.VectorSubcoreMesh(core_axis_name="c", subcore_axis_name="s")

    @functools.partial(
        pl.kernel,
        mesh=mesh,
        out_type=jax.ShapeDtypeStruct((HIST, DB, BATCH // _BB, 8, _BB), jnp.float32),
        scratch_types=[
            pltpu.VMEM((b_w, HIST), jnp.int32),
            pltpu.VMEM((HP, b_w + 1), jnp.int32),
            [pltpu.VMEM((_BB, D), jnp.float32) for _ in range(2)],
            [pltpu.VMEM((DB, 8, _BB + 1), jnp.float32) for _ in range(2)],
            [pltpu.SemaphoreType.DMA for _ in range(2)],
            [pltpu.SemaphoreType.DMA for _ in range(2)],
        ],
        compiler_params=pltpu.CompilerParams(
            use_tc_tiling_on_sc=False, needs_layout_passes=False
        ),
    )
    def gather_kernel(feat_hbm, table_hbm, out_hbm, idx_v, idx_t, grows, tbufs,
                      gsems, ssems):
        wid = lax.axis_index("s") * NC + lax.axis_index("c")
        base = wid * b_w
        lanes = lax.iota(jnp.int32, L)
        pltpu.sync_copy(feat_hbm.at[pl.ds(base, b_w)], idx_v)

        offs = [k * L for k in range(HIST // L)]
        if HIST % L:
            offs.append(HIST - L)

        def tr_idx(b, carry):
            bvec = jnp.full((L,), b, jnp.int32)
            for o in offs:
                v = idx_v[b, pl.ds(o, L)]
                plsc.store_scatter(idx_t, [o + lanes, bvec], v)
            return carry

        lax.fori_loop(0, b_w, tr_idx, 0)

        dbvs = [(k * L + lanes) // 8 for k in range(D // L)]
        dsvs = [(k * L + lanes) % 8 for k in range(D // L)]

        def start_gather(u, p):
            h, jb = u // njb, u % njb
            return pltpu.async_copy(
                table_hbm.at[idx_t.at[h, pl.ds(jb * _BB, _BB)]], grows[p],
                gsems[p])

        def wait_gather(p):
            pltpu.make_async_copy(
                table_hbm.at[idx_t.at[0, pl.ds(0, _BB)]], grows[p], gsems[p]
            ).wait()

        def transpose_unit(p):
            def tr(i, carry):
                for bb in range(8):
                    b = i * 8 + bb
                    bvec = jnp.full((L,), b, jnp.int32)
                    for k in range(D // L):
                        v = grows[p][b, pl.ds(k * L, L)]
                        plsc.store_scatter(tbufs[p], [dbvs[k], dsvs[k], bvec], v)
                return carry

            lax.fori_loop(0, _BB // 8, tr, 0)

        def start_store(u, p):
            h, jb = u // njb, u % njb
            return pltpu.async_copy(
                tbufs[p].at[:, :, pl.ds(0, _BB)],
                out_hbm.at[h, :, wid * njb + jb], ssems[p])

        def wait_store(p):
            pltpu.make_async_copy(
                tbufs[p].at[:, :, pl.ds(0, _BB)], out_hbm.at[0, :, 0], ssems[p]
            ).wait()

        start_gather(0, 0)

        def pair_body(q, carry):
            u0 = 2 * q
            wait_gather(0)
            start_gather(u0 + 1, 1)
            transpose_unit(0)
            lax.cond(q > 0, lambda: wait_store(0), lambda: None)
            start_store(u0, 0)
            wait_gather(1)
            lax.cond(u0 + 2 < n_units,
                     lambda: (start_gather(u0 + 2, 0), None)[1], lambda: None)
            transpose_unit(1)
            lax.cond(q > 0, lambda: wait_store(1), lambda: None)
            start_store(u0 + 1, 1)
            return carry

        lax.fori_loop(0, n_units // 2, pair_body, 0)
        wait_store(0)
        wait_store(1)

    return gather_kernel


def kernel(feature, table):
    batch, hist = feature.shape
    dim = table.shape[1]
    out_p = _make_gather(table.shape[0], dim, batch, hist)(feature, table)
    return jnp.transpose(out_p, (2, 4, 0, 1, 3)).reshape(batch, hist, dim)
